# baseline (speedup 1.0000x reference)
import functools

import jax
import jax.numpy as jnp
from jax import lax
from jax.experimental import pallas as pl
from jax.experimental.pallas import tpu as pltpu
from jax.experimental.pallas import tpu_sc as plsc

N = 10000
D = 128
F = 64
E = 320000

NC = 2
NS = 16
NW = NC * NS
CH = 128
CPT = 80
EPT = CPT * CH
EPAD = EPT * NW
NPAD = 10240
SLICE = NPAD // NS
G16 = CH // 16
NBUF = 4

_mesh = plsc.VectorSubcoreMesh(
    core_axis_name="c", subcore_axis_name="s", num_cores=NC, num_subcores=NS)
_params = pltpu.CompilerParams(
    needs_layout_passes=False, use_tc_tiling_on_sc=False)


def _rsqrt16(x):
    i = lax.bitcast_convert_type(x, jnp.int32)
    i = jnp.int32(0x5F3759DF) - lax.shift_right_arithmetic(i, 1)
    y = lax.bitcast_convert_type(i, jnp.float32)
    for _ in range(3):
        y = y * (1.5 - 0.5 * x * y * y)
    return y


@functools.partial(
    pl.kernel,
    out_type=jax.ShapeDtypeStruct((NC, NPAD), jnp.float32),
    mesh=_mesh,
    compiler_params=_params,
    scratch_types=[
        pltpu.VMEM((CPT, CH), jnp.int32),
        pltpu.VMEM((CPT, CH), jnp.int32),
        pltpu.VMEM((CPT, CH), jnp.float32),
        pltpu.VMEM((SLICE,), jnp.float32),
        pltpu.SemaphoreType.DMA,
        pltpu.VMEM_SHARED((NPAD,), jnp.float32),
    ],
)
def _sc_degree(ei, w, deg_out, src_a, dst_a, w_a, zbuf, ssem, deg_sh):
    cid = lax.axis_index("c")
    sid = lax.axis_index("s")
    wid = cid * NS + sid

    pltpu.sync_copy(ei.at[0, wid], src_a)
    pltpu.sync_copy(ei.at[1, wid], dst_a)
    pltpu.sync_copy(w.at[wid], w_a)

    @pl.loop(0, SLICE // 16)
    def _(i):
        zbuf[pl.ds(i * 16, 16)] = jnp.zeros((16,), jnp.float32)

    pltpu.sync_copy(zbuf, deg_sh.at[pl.ds(sid * SLICE, SLICE)])
    plsc.subcore_barrier()

    WIN = 8

    @pl.loop(0, CPT)
    def _(c):
        @pl.loop(0, G16)
        def _(g):
            sl = pl.ds(g * 16, 16)
            w_a[c, sl] = jnp.where(src_a[c, sl] == dst_a[c, sl],
                                   0.0, w_a[c, sl])

        @pl.when(c >= WIN)
        def _():
            pltpu.make_async_copy(
                w_a.at[0], deg_sh.at[src_a.at[0]], ssem).wait()

        pltpu.async_copy(w_a.at[c], deg_sh.at[src_a.at[c]], ssem, add=True)

    @pl.loop(0, WIN)
    def _(i):
        pltpu.make_async_copy(w_a.at[0], deg_sh.at[src_a.at[0]], ssem).wait()

    plsc.subcore_barrier()
    pltpu.sync_copy(deg_sh.at[pl.ds(sid * SLICE, SLICE)],
                    deg_out.at[cid, pl.ds(sid * SLICE, SLICE)])


def _zero_acc(rows, t_sh, sid):
    @pl.loop(0, CH)
    def _(r):
        for l in range(F // 16):
            rows[r, pl.ds(l * 16, 16)] = jnp.zeros((16,), jnp.float32)

    r0 = sid * SLICE
    for k in range(SLICE // CH):
        pltpu.sync_copy(rows, t_sh.at[pl.ds(r0 + k * CH, CH)])


def _scale_rows(rows, lap_a, c):
    @plsc.parallel_loop(0, G16, step=1, unroll=4)
    def _(g):
        lv = lap_a[c, pl.ds(g * 16, 16)]
        for j in range(16):
            lw = lv[j]
            r = g * 16 + j
            for l in range(F // 16):
                sl = pl.ds(l * 16, 16)
                rows[r, sl] = rows[r, sl] * lw


def _prop_pipeline(y_src, src_a, dst_a, lap_a, rows_bufs, gsems, t_sh,
                   per_chunk):
    nbuf = len(rows_bufs)
    assert CPT % nbuf == 0

    def issue_g(c, rows, sem):
        pltpu.async_copy(y_src.at[src_a.at[c]], rows, sem)

    def wait_g(rows, sem):
        pltpu.make_async_copy(y_src.at[src_a.at[0]], rows, sem).wait()

    for b in range(nbuf):
        issue_g(b, rows_bufs[b], gsems[b])

    @pl.loop(0, CPT // nbuf)
    def _(k):
        for b in range(nbuf):
            c = nbuf * k + b
            per_chunk(c)
            wait_g(rows_bufs[b], gsems[b])
            _scale_rows(rows_bufs[b], lap_a, c)
            pltpu.sync_copy(rows_bufs[b], t_sh.at[dst_a.at[c]], add=True)

            @pl.when(c + nbuf < CPT)
            def _():
                issue_g(c + nbuf, rows_bufs[b], gsems[b])


@functools.partial(
    pl.kernel,
    out_type=(jax.ShapeDtypeStruct((NC, NPAD, F), jnp.float32),
              jax.ShapeDtypeStruct((NW, CPT, CH), jnp.float32)),
    mesh=_mesh,
    compiler_params=_params,
    scratch_types=[
        pltpu.VMEM((CPT, CH), jnp.int32),
        pltpu.VMEM((CPT, CH), jnp.int32),
        pltpu.VMEM((CPT, CH), jnp.float32),
        pltpu.VMEM((CPT, CH), jnp.float32),
        [pltpu.VMEM((CH, F), jnp.float32)] * NBUF,
        pltpu.VMEM((NPAD,), jnp.float32),
        pltpu.VMEM((SLICE,), jnp.float32),
        pltpu.VMEM((SLICE,), jnp.float32),
        [pltpu.SemaphoreType.DMA] * NBUF,
        pltpu.VMEM_SHARED((NPAD, F), jnp.float32),
        pltpu.VMEM_SHARED((NPAD,), jnp.float32),
    ],
)
def _sc_prop1(ei, w, deg_part, y, t_out, lap_out,
              src_a, dst_a, w_a, lap_a, rows_bufs, dis_t, dga, dgb,
              gsems, t_sh, dis_sh):
    cid = lax.axis_index("c")
    sid = lax.axis_index("s")
    wid = cid * NS + sid

    pltpu.sync_copy(ei.at[0, wid], src_a)
    pltpu.sync_copy(ei.at[1, wid], dst_a)
    pltpu.sync_copy(w.at[wid], w_a)

    pltpu.sync_copy(deg_part.at[0, pl.ds(sid * SLICE, SLICE)], dga)
    pltpu.sync_copy(deg_part.at[1, pl.ds(sid * SLICE, SLICE)], dgb)

    @pl.loop(0, SLICE // 16)
    def _(i):
        sl = pl.ds(i * 16, 16)
        dsum = dga[sl] + dgb[sl]
        pos = dsum > 0.0
        dsafe = jnp.where(pos, dsum, 1.0)
        dga[sl] = jnp.where(pos, _rsqrt16(dsafe), 0.0)

    pltpu.sync_copy(dga, dis_sh.at[pl.ds(sid * SLICE, SLICE)])
    _zero_acc(rows_bufs[0], t_sh, sid)
    plsc.subcore_barrier()
    pltpu.sync_copy(dis_sh, dis_t)

    def make_lap(c):
        for g in range(G16):
            sl = pl.ds(g * 16, 16)
            s = src_a[c, sl]
            d = dst_a[c, sl]
            wv = jnp.where(s == d, 0.0, w_a[c, sl])
            dsv = plsc.load_gather(dis_t, [s])
            ddv = plsc.load_gather(dis_t, [d])
            lap_a[c, sl] = -(dsv * wv) * ddv

    _prop_pipeline(y, src_a, dst_a, lap_a, rows_bufs, gsems, t_sh, make_lap)

    pltpu.sync_copy(lap_a, lap_out.at[wid])
    plsc.subcore_barrier()
    pltpu.sync_copy(t_sh.at[pl.ds(sid * SLICE, SLICE)],
                    t_out.at[cid, pl.ds(sid * SLICE, SLICE)])


@functools.partial(
    pl.kernel,
    out_type=jax.ShapeDtypeStruct((NC, NPAD, F), jnp.float32),
    mesh=_mesh,
    compiler_params=_params,
    scratch_types=[
        pltpu.VMEM((CPT, CH), jnp.int32),
        pltpu.VMEM((CPT, CH), jnp.int32),
        pltpu.VMEM((CPT, CH), jnp.float32),
        [pltpu.VMEM((CH, F), jnp.float32)] * 2,
        [pltpu.SemaphoreType.DMA] * 2,
        pltpu.VMEM_SHARED((NPAD, F), jnp.float32),
        pltpu.VMEM_SHARED((N, F), jnp.float32),
    ],
)
def _sc_prop2(ei, lap, y, t_out,
              src_a, dst_a, lap_a, rows_bufs, gsems, t_sh, y_sh):
    cid = lax.axis_index("c")
    sid = lax.axis_index("s")
    wid = cid * NS + sid

    pltpu.sync_copy(ei.at[0, wid], src_a)
    pltpu.sync_copy(ei.at[1, wid], dst_a)
    pltpu.sync_copy(lap.at[wid], lap_a)

    _zero_acc(rows_bufs[0], t_sh, sid)

    @pl.when(sid == 0)
    def _():
        pltpu.sync_copy(y, y_sh)

    plsc.subcore_barrier()

    _prop_pipeline(y_sh, src_a, dst_a, lap_a, rows_bufs, gsems, t_sh,
                   lambda c: None)

    plsc.subcore_barrier()
    pltpu.sync_copy(t_sh.at[pl.ds(sid * SLICE, SLICE)],
                    t_out.at[cid, pl.ds(sid * SLICE, SLICE)])


_RB = 2000
_RBM = 1280


def _mm_body(x_ref, w_ref, o_ref):
    o_ref[...] = jnp.dot(x_ref[...], w_ref[...],
                         precision=lax.Precision.HIGHEST,
                         preferred_element_type=jnp.float32)


def _comb_body(b_ref, t_ref, o_ref):
    o_ref[...] = b_ref[...] + 2.0 * (t_ref[0] + t_ref[1])


def _fin_body(a_ref, u_ref, bc_ref, wl_ref, bl_ref, o_ref):
    h = a_ref[...] + u_ref[0] + u_ref[1] + bc_ref[...]
    h = jnp.maximum(h, 0.0)
    o_ref[...] = jnp.dot(h, wl_ref[...],
                         precision=lax.Precision.HIGHEST,
                         preferred_element_type=jnp.float32) + bl_ref[...]


def kernel(x, edge_index, edge_weight, W0, W1, W2, b_cheb, W_lin, b_lin):
    Wc = jnp.concatenate([W0 - W2, W1, W2], axis=1)
    x_p = jnp.pad(x, ((0, NPAD - N), (0, 0)))

    abc = pl.pallas_call(
        _mm_body,
        grid=(NPAD // _RBM,),
        in_specs=[pl.BlockSpec((_RBM, D), lambda i: (i, 0)),
                  pl.BlockSpec((D, 3 * F), lambda i: (0, 0))],
        out_specs=pl.BlockSpec((_RBM, 3 * F), lambda i: (i, 0)),
        out_shape=jax.ShapeDtypeStruct((NPAD, 3 * F), jnp.float32),
    )(x_p, Wc)
    A = abc[:, :F]
    B = abc[:, F:2 * F]
    C = abc[:, 2 * F:]

    pad = EPAD - E
    ei_p = jnp.pad(edge_index, ((0, 0), (0, pad))).reshape(2, NW, CPT, CH)
    w_p = jnp.pad(edge_weight, ((0, pad),)).reshape(NW, CPT, CH)

    deg_part = _sc_degree(ei_p, w_p)
    t_part, lap = _sc_prop1(ei_p, w_p, deg_part, C)

    M = pl.pallas_call(
        _comb_body,
        grid=(N // _RB,),
        in_specs=[pl.BlockSpec((_RB, F), lambda i: (i, 0)),
                  pl.BlockSpec((NC, _RB, F), lambda i: (0, i, 0))],
        out_specs=pl.BlockSpec((_RB, F), lambda i: (i, 0)),
        out_shape=jax.ShapeDtypeStruct((N, F), jnp.float32),
    )(B, t_part)

    u_part = _sc_prop2(ei_p, lap, M)

    out = pl.pallas_call(
        _fin_body,
        grid=(N // _RB,),
        in_specs=[pl.BlockSpec((_RB, F), lambda i: (i, 0)),
                  pl.BlockSpec((NC, _RB, F), lambda i: (0, i, 0)),
                  pl.BlockSpec((1, F), lambda i: (0, 0)),
                  pl.BlockSpec((F, 1), lambda i: (0, 0)),
                  pl.BlockSpec((1, 1), lambda i: (0, 0))],
        out_specs=pl.BlockSpec((_RB, 1), lambda i: (i, 0)),
        out_shape=jax.ShapeDtypeStruct((N, 1), jnp.float32),
    )(A, u_part, b_cheb.reshape(1, F), W_lin, b_lin.reshape(1, 1))
    return out


# speedup vs baseline: 1.0083x; 1.0083x over previous
import functools

import jax
import jax.numpy as jnp
from jax import lax
from jax.experimental import pallas as pl
from jax.experimental.pallas import tpu as pltpu
from jax.experimental.pallas import tpu_sc as plsc

N = 10000
D = 128
F = 64
E = 320000

NC = 2
NS = 16
NW = NC * NS
CH = 128
CPT = 80
EPT = CPT * CH
EPAD = EPT * NW
NPAD = 10240
SLICE = NPAD // NS
G16 = CH // 16
NBUF = 2

_mesh = plsc.VectorSubcoreMesh(
    core_axis_name="c", subcore_axis_name="s", num_cores=NC, num_subcores=NS)
_params = pltpu.CompilerParams(
    needs_layout_passes=False, use_tc_tiling_on_sc=False)


def _rsqrt16(x):
    i = lax.bitcast_convert_type(x, jnp.int32)
    i = jnp.int32(0x5F3759DF) - lax.shift_right_arithmetic(i, 1)
    y = lax.bitcast_convert_type(i, jnp.float32)
    for _ in range(3):
        y = y * (1.5 - 0.5 * x * y * y)
    return y


@functools.partial(
    pl.kernel,
    out_type=jax.ShapeDtypeStruct((NC, NPAD), jnp.float32),
    mesh=_mesh,
    compiler_params=_params,
    scratch_types=[
        pltpu.VMEM((CPT, CH), jnp.int32),
        pltpu.VMEM((CPT, CH), jnp.int32),
        pltpu.VMEM((CPT, CH), jnp.float32),
        pltpu.VMEM((SLICE,), jnp.float32),
        pltpu.SemaphoreType.DMA,
        pltpu.VMEM_SHARED((NPAD,), jnp.float32),
    ],
)
def _sc_degree(ei, w, deg_out, src_a, dst_a, w_a, zbuf, ssem, deg_sh):
    cid = lax.axis_index("c")
    sid = lax.axis_index("s")
    wid = cid * NS + sid

    pltpu.sync_copy(ei.at[0, wid], src_a)
    pltpu.sync_copy(ei.at[1, wid], dst_a)
    pltpu.sync_copy(w.at[wid], w_a)

    @pl.loop(0, SLICE // 16)
    def _(i):
        zbuf[pl.ds(i * 16, 16)] = jnp.zeros((16,), jnp.float32)

    pltpu.sync_copy(zbuf, deg_sh.at[pl.ds(sid * SLICE, SLICE)])
    plsc.subcore_barrier()

    WIN = 8

    @pl.loop(0, CPT)
    def _(c):
        @pl.loop(0, G16)
        def _(g):
            sl = pl.ds(g * 16, 16)
            w_a[c, sl] = jnp.where(src_a[c, sl] == dst_a[c, sl],
                                   0.0, w_a[c, sl])

        @pl.when(c >= WIN)
        def _():
            pltpu.make_async_copy(
                w_a.at[0], deg_sh.at[src_a.at[0]], ssem).wait()

        pltpu.async_copy(w_a.at[c], deg_sh.at[src_a.at[c]], ssem, add=True)

    @pl.loop(0, WIN)
    def _(i):
        pltpu.make_async_copy(w_a.at[0], deg_sh.at[src_a.at[0]], ssem).wait()

    plsc.subcore_barrier()
    pltpu.sync_copy(deg_sh.at[pl.ds(sid * SLICE, SLICE)],
                    deg_out.at[cid, pl.ds(sid * SLICE, SLICE)])


def _zero_acc(rows, t_sh, sid):
    @pl.loop(0, CH)
    def _(r):
        for l in range(F // 16):
            rows[r, pl.ds(l * 16, 16)] = jnp.zeros((16,), jnp.float32)

    r0 = sid * SLICE
    for k in range(SLICE // CH):
        pltpu.sync_copy(rows, t_sh.at[pl.ds(r0 + k * CH, CH)])


def _scale_rows(rows, lap_a, c):
    @plsc.parallel_loop(0, G16, step=1, unroll=4)
    def _(g):
        lv = lap_a[c, pl.ds(g * 16, 16)]
        for j in range(16):
            lw = lv[j]
            r = g * 16 + j
            for l in range(F // 16):
                sl = pl.ds(l * 16, 16)
                rows[r, sl] = rows[r, sl] * lw


def _prop_pipeline(y_src, src_a, dst_a, lap_a, rows_bufs, gsems, t_sh,
                   per_chunk):
    nbuf = len(rows_bufs)
    assert CPT % nbuf == 0

    def issue_g(c, rows, sem):
        pltpu.async_copy(y_src.at[src_a.at[c]], rows, sem)

    def wait_g(rows, sem):
        pltpu.make_async_copy(y_src.at[src_a.at[0]], rows, sem).wait()

    for b in range(nbuf):
        issue_g(b, rows_bufs[b], gsems[b])

    @pl.loop(0, CPT // nbuf)
    def _(k):
        for b in range(nbuf):
            c = nbuf * k + b
            per_chunk(c)
            wait_g(rows_bufs[b], gsems[b])
            _scale_rows(rows_bufs[b], lap_a, c)
            pltpu.sync_copy(rows_bufs[b], t_sh.at[dst_a.at[c]], add=True)

            @pl.when(c + nbuf < CPT)
            def _():
                issue_g(c + nbuf, rows_bufs[b], gsems[b])


@functools.partial(
    pl.kernel,
    out_type=(jax.ShapeDtypeStruct((NC, NPAD, F), jnp.float32),
              jax.ShapeDtypeStruct((NW, CPT, CH), jnp.float32)),
    mesh=_mesh,
    compiler_params=_params,
    scratch_types=[
        pltpu.VMEM((CPT, CH), jnp.int32),
        pltpu.VMEM((CPT, CH), jnp.int32),
        pltpu.VMEM((CPT, CH), jnp.float32),
        pltpu.VMEM((CPT, CH), jnp.float32),
        [pltpu.VMEM((CH, F), jnp.float32)] * NBUF,
        pltpu.VMEM((NPAD,), jnp.float32),
        pltpu.VMEM((SLICE,), jnp.float32),
        pltpu.VMEM((SLICE,), jnp.float32),
        [pltpu.SemaphoreType.DMA] * NBUF,
        pltpu.VMEM_SHARED((NPAD, F), jnp.float32),
        pltpu.VMEM_SHARED((NPAD,), jnp.float32),
    ],
)
def _sc_prop1(ei, w, deg_part, y, t_out, lap_out,
              src_a, dst_a, w_a, lap_a, rows_bufs, dis_t, dga, dgb,
              gsems, t_sh, dis_sh):
    cid = lax.axis_index("c")
    sid = lax.axis_index("s")
    wid = cid * NS + sid

    pltpu.sync_copy(ei.at[0, wid], src_a)
    pltpu.sync_copy(ei.at[1, wid], dst_a)
    pltpu.sync_copy(w.at[wid], w_a)

    pltpu.sync_copy(deg_part.at[0, pl.ds(sid * SLICE, SLICE)], dga)
    pltpu.sync_copy(deg_part.at[1, pl.ds(sid * SLICE, SLICE)], dgb)

    @pl.loop(0, SLICE // 16)
    def _(i):
        sl = pl.ds(i * 16, 16)
        dsum = dga[sl] + dgb[sl]
        pos = dsum > 0.0
        dsafe = jnp.where(pos, dsum, 1.0)
        dga[sl] = jnp.where(pos, _rsqrt16(dsafe), 0.0)

    pltpu.sync_copy(dga, dis_sh.at[pl.ds(sid * SLICE, SLICE)])
    _zero_acc(rows_bufs[0], t_sh, sid)
    plsc.subcore_barrier()
    pltpu.sync_copy(dis_sh, dis_t)

    def make_lap(c):
        for g in range(G16):
            sl = pl.ds(g * 16, 16)
            s = src_a[c, sl]
            d = dst_a[c, sl]
            wv = jnp.where(s == d, 0.0, w_a[c, sl])
            dsv = plsc.load_gather(dis_t, [s])
            ddv = plsc.load_gather(dis_t, [d])
            lap_a[c, sl] = -(dsv * wv) * ddv

    _prop_pipeline(y, src_a, dst_a, lap_a, rows_bufs, gsems, t_sh, make_lap)

    pltpu.sync_copy(lap_a, lap_out.at[wid])
    plsc.subcore_barrier()
    pltpu.sync_copy(t_sh.at[pl.ds(sid * SLICE, SLICE)],
                    t_out.at[cid, pl.ds(sid * SLICE, SLICE)])


@functools.partial(
    pl.kernel,
    out_type=jax.ShapeDtypeStruct((NC, NPAD, F), jnp.float32),
    mesh=_mesh,
    compiler_params=_params,
    scratch_types=[
        pltpu.VMEM((CPT, CH), jnp.int32),
        pltpu.VMEM((CPT, CH), jnp.int32),
        pltpu.VMEM((CPT, CH), jnp.float32),
        [pltpu.VMEM((CH, F), jnp.float32)] * 2,
        [pltpu.SemaphoreType.DMA] * 2,
        pltpu.VMEM_SHARED((NPAD, F), jnp.float32),
        pltpu.VMEM_SHARED((N, F), jnp.float32),
    ],
)
def _sc_prop2(ei, lap, y, t_out,
              src_a, dst_a, lap_a, rows_bufs, gsems, t_sh, y_sh):
    cid = lax.axis_index("c")
    sid = lax.axis_index("s")
    wid = cid * NS + sid

    pltpu.sync_copy(ei.at[0, wid], src_a)
    pltpu.sync_copy(ei.at[1, wid], dst_a)
    pltpu.sync_copy(lap.at[wid], lap_a)

    _zero_acc(rows_bufs[0], t_sh, sid)

    @pl.when(sid == 0)
    def _():
        pltpu.sync_copy(y, y_sh)

    plsc.subcore_barrier()

    _prop_pipeline(y_sh, src_a, dst_a, lap_a, rows_bufs, gsems, t_sh,
                   lambda c: None)

    plsc.subcore_barrier()
    pltpu.sync_copy(t_sh.at[pl.ds(sid * SLICE, SLICE)],
                    t_out.at[cid, pl.ds(sid * SLICE, SLICE)])


_RB = 2000
_RBM = 1280


def _mm_body(x_ref, w_ref, o_ref):
    o_ref[...] = jnp.dot(x_ref[...], w_ref[...],
                         precision=lax.Precision.HIGHEST,
                         preferred_element_type=jnp.float32)


def _comb_body(b_ref, t_ref, o_ref):
    o_ref[...] = b_ref[...] + 2.0 * (t_ref[0] + t_ref[1])


def _fin_body(a_ref, u_ref, bc_ref, wl_ref, bl_ref, o_ref):
    h = a_ref[...] + u_ref[0] + u_ref[1] + bc_ref[...]
    h = jnp.maximum(h, 0.0)
    o_ref[...] = jnp.dot(h, wl_ref[...],
                         precision=lax.Precision.HIGHEST,
                         preferred_element_type=jnp.float32) + bl_ref[...]


def kernel(x, edge_index, edge_weight, W0, W1, W2, b_cheb, W_lin, b_lin):
    Wc = jnp.concatenate([W0 - W2, W1, W2], axis=1)
    x_p = jnp.pad(x, ((0, NPAD - N), (0, 0)))

    abc = pl.pallas_call(
        _mm_body,
        grid=(NPAD // _RBM,),
        in_specs=[pl.BlockSpec((_RBM, D), lambda i: (i, 0)),
                  pl.BlockSpec((D, 3 * F), lambda i: (0, 0))],
        out_specs=pl.BlockSpec((_RBM, 3 * F), lambda i: (i, 0)),
        out_shape=jax.ShapeDtypeStruct((NPAD, 3 * F), jnp.float32),
    )(x_p, Wc)
    A = abc[:, :F]
    B = abc[:, F:2 * F]
    C = abc[:, 2 * F:]

    pad = EPAD - E
    ei_p = jnp.pad(edge_index, ((0, 0), (0, pad))).reshape(2, NW, CPT, CH)
    w_p = jnp.pad(edge_weight, ((0, pad),)).reshape(NW, CPT, CH)

    deg_part = _sc_degree(ei_p, w_p)
    t_part, lap = _sc_prop1(ei_p, w_p, deg_part, C)

    M = pl.pallas_call(
        _comb_body,
        grid=(N // _RB,),
        in_specs=[pl.BlockSpec((_RB, F), lambda i: (i, 0)),
                  pl.BlockSpec((NC, _RB, F), lambda i: (0, i, 0))],
        out_specs=pl.BlockSpec((_RB, F), lambda i: (i, 0)),
        out_shape=jax.ShapeDtypeStruct((N, F), jnp.float32),
    )(B, t_part)

    u_part = _sc_prop2(ei_p, lap, M)

    out = pl.pallas_call(
        _fin_body,
        grid=(N // _RB,),
        in_specs=[pl.BlockSpec((_RB, F), lambda i: (i, 0)),
                  pl.BlockSpec((NC, _RB, F), lambda i: (0, i, 0)),
                  pl.BlockSpec((1, F), lambda i: (0, 0)),
                  pl.BlockSpec((F, 1), lambda i: (0, 0)),
                  pl.BlockSpec((1, 1), lambda i: (0, 0))],
        out_specs=pl.BlockSpec((_RB, 1), lambda i: (i, 0)),
        out_shape=jax.ShapeDtypeStruct((N, 1), jnp.float32),
    )(A, u_part, b_cheb.reshape(1, F), W_lin, b_lin.reshape(1, 1))
    return out


# speedup vs baseline: 1.0679x; 1.0591x over previous
import functools

import jax
import jax.numpy as jnp
from jax import lax
from jax.experimental import pallas as pl
from jax.experimental.pallas import tpu as pltpu
from jax.experimental.pallas import tpu_sc as plsc

N = 10000
D = 128
F = 64
E = 320000

NC = 2
NS = 16
NW = NC * NS
CH = 128
CPT = 80
EPT = CPT * CH
EPAD = EPT * NW
NPAD = 10240
SLICE = NPAD // NS
G16 = CH // 16

_mesh = plsc.VectorSubcoreMesh(
    core_axis_name="c", subcore_axis_name="s", num_cores=NC, num_subcores=NS)
_params = pltpu.CompilerParams(
    needs_layout_passes=False, use_tc_tiling_on_sc=False)


def _rsqrt16(x):
    i = lax.bitcast_convert_type(x, jnp.int32)
    i = jnp.int32(0x5F3759DF) - lax.shift_right_arithmetic(i, 1)
    y = lax.bitcast_convert_type(i, jnp.float32)
    for _ in range(3):
        y = y * (1.5 - 0.5 * x * y * y)
    return y


@functools.partial(
    pl.kernel,
    out_type=jax.ShapeDtypeStruct((NC, NPAD), jnp.float32),
    mesh=_mesh,
    compiler_params=_params,
    scratch_types=[
        pltpu.VMEM((CPT, CH), jnp.int32),
        pltpu.VMEM((CPT, CH), jnp.int32),
        pltpu.VMEM((CPT, CH), jnp.float32),
        pltpu.VMEM((SLICE,), jnp.float32),
        pltpu.SemaphoreType.DMA,
        pltpu.VMEM_SHARED((NPAD,), jnp.float32),
    ],
)
def _sc_degree(ei, w, deg_out, src_a, dst_a, w_a, zbuf, ssem, deg_sh):
    cid = lax.axis_index("c")
    sid = lax.axis_index("s")
    wid = cid * NS + sid

    pltpu.sync_copy(ei.at[0, wid], src_a)
    pltpu.sync_copy(ei.at[1, wid], dst_a)
    pltpu.sync_copy(w.at[wid], w_a)

    @pl.loop(0, SLICE // 16)
    def _(i):
        zbuf[pl.ds(i * 16, 16)] = jnp.zeros((16,), jnp.float32)

    pltpu.sync_copy(zbuf, deg_sh.at[pl.ds(sid * SLICE, SLICE)])
    plsc.subcore_barrier()

    WIN = 8

    @pl.loop(0, CPT)
    def _(c):
        @pl.loop(0, G16)
        def _(g):
            sl = pl.ds(g * 16, 16)
            w_a[c, sl] = jnp.where(src_a[c, sl] == dst_a[c, sl],
                                   0.0, w_a[c, sl])

        @pl.when(c >= WIN)
        def _():
            pltpu.make_async_copy(
                w_a.at[0], deg_sh.at[src_a.at[0]], ssem).wait()

        pltpu.async_copy(w_a.at[c], deg_sh.at[src_a.at[c]], ssem, add=True)

    @pl.loop(0, WIN)
    def _(i):
        pltpu.make_async_copy(w_a.at[0], deg_sh.at[src_a.at[0]], ssem).wait()

    plsc.subcore_barrier()
    pltpu.sync_copy(deg_sh.at[pl.ds(sid * SLICE, SLICE)],
                    deg_out.at[cid, pl.ds(sid * SLICE, SLICE)])


def _zero_acc(rows, t_sh, sid):
    @pl.loop(0, CH)
    def _(r):
        for l in range(F // 16):
            rows[r, pl.ds(l * 16, 16)] = jnp.zeros((16,), jnp.float32)

    r0 = sid * SLICE
    for k in range(SLICE // CH):
        pltpu.sync_copy(rows, t_sh.at[pl.ds(r0 + k * CH, CH)])


def _scale_rows(rows, lap_a, c):
    @plsc.parallel_loop(0, G16, step=1, unroll=4)
    def _(g):
        lv = lap_a[c, pl.ds(g * 16, 16)]
        for j in range(16):
            lw = lv[j]
            r = g * 16 + j
            for l in range(F // 16):
                sl = pl.ds(l * 16, 16)
                rows[r, sl] = rows[r, sl] * lw


def _prop_pipeline(y_sh, src_a, dst_a, lap_a, rows_ab, gsems, ssems, t_sh,
                   make_lap):
    rows_a, rows_b = rows_ab
    gsem_a, gsem_b = gsems
    ssem_a, ssem_b = ssems

    def issue_g(c, rows, sem):
        pltpu.async_copy(y_sh.at[src_a.at[c]], rows, sem)

    def wait_g(rows, sem):
        pltpu.make_async_copy(y_sh.at[src_a.at[0]], rows, sem).wait()

    def issue_s(c, rows, sem):
        pltpu.async_copy(rows, t_sh.at[dst_a.at[c]], sem, add=True)

    def wait_s(rows, sem):
        pltpu.make_async_copy(rows, t_sh.at[dst_a.at[0]], sem).wait()

    issue_g(0, rows_a, gsem_a)
    issue_g(1, rows_b, gsem_b)

    @pl.loop(0, CPT // 2)
    def _(k):
        c0 = 2 * k
        c1 = c0 + 1
        make_lap(c0)
        wait_g(rows_a, gsem_a)
        _scale_rows(rows_a, lap_a, c0)
        pltpu.sync_copy(rows_a, t_sh.at[dst_a.at[c0]], add=True)

        @pl.when(c0 + 2 < CPT)
        def _():
            issue_g(c0 + 2, rows_a, gsem_a)

        make_lap(c1)
        wait_g(rows_b, gsem_b)
        _scale_rows(rows_b, lap_a, c1)
        pltpu.sync_copy(rows_b, t_sh.at[dst_a.at[c1]], add=True)

        @pl.when(c1 + 2 < CPT)
        def _():
            issue_g(c1 + 2, rows_b, gsem_b)


@functools.partial(
    pl.kernel,
    out_type=(jax.ShapeDtypeStruct((NC, NPAD, F), jnp.float32),
              jax.ShapeDtypeStruct((NW, CPT, CH), jnp.float32)),
    mesh=_mesh,
    compiler_params=_params,
    scratch_types=[
        pltpu.VMEM((CPT, CH), jnp.int32),
        pltpu.VMEM((CPT, CH), jnp.int32),
        pltpu.VMEM((CPT, CH), jnp.float32),
        pltpu.VMEM((CPT, CH), jnp.float32),
        pltpu.VMEM((CH, F), jnp.float32),
        pltpu.VMEM((CH, F), jnp.float32),
        pltpu.VMEM((NPAD,), jnp.float32),
        pltpu.VMEM((SLICE,), jnp.float32),
        pltpu.VMEM((SLICE,), jnp.float32),
        pltpu.SemaphoreType.DMA,
        pltpu.SemaphoreType.DMA,
        pltpu.SemaphoreType.DMA,
        pltpu.SemaphoreType.DMA,
        pltpu.VMEM_SHARED((NPAD, F), jnp.float32),
        pltpu.VMEM_SHARED((NPAD,), jnp.float32),
    ],
)
def _sc_prop1(ei, w, deg_part, y, t_out, lap_out,
              src_a, dst_a, w_a, lap_a, rows_a, rows_b, dis_t, dga, dgb,
              gsem_a, gsem_b, ssem_a, ssem_b, t_sh, dis_sh):
    cid = lax.axis_index("c")
    sid = lax.axis_index("s")
    wid = cid * NS + sid

    pltpu.sync_copy(ei.at[0, wid], src_a)
    pltpu.sync_copy(ei.at[1, wid], dst_a)
    pltpu.sync_copy(w.at[wid], w_a)

    pltpu.sync_copy(deg_part.at[0, pl.ds(sid * SLICE, SLICE)], dga)
    pltpu.sync_copy(deg_part.at[1, pl.ds(sid * SLICE, SLICE)], dgb)

    @pl.loop(0, SLICE // 16)
    def _(i):
        sl = pl.ds(i * 16, 16)
        dsum = dga[sl] + dgb[sl]
        pos = dsum > 0.0
        dsafe = jnp.where(pos, dsum, 1.0)
        dga[sl] = jnp.where(pos, _rsqrt16(dsafe), 0.0)

    pltpu.sync_copy(dga, dis_sh.at[pl.ds(sid * SLICE, SLICE)])
    _zero_acc(rows_a, t_sh, sid)
    plsc.subcore_barrier()
    pltpu.sync_copy(dis_sh, dis_t)

    def make_lap(c):
        for g in range(G16):
            sl = pl.ds(g * 16, 16)
            s = src_a[c, sl]
            d = dst_a[c, sl]
            wv = jnp.where(s == d, 0.0, w_a[c, sl])
            dsv = plsc.load_gather(dis_t, [s])
            ddv = plsc.load_gather(dis_t, [d])
            lap_a[c, sl] = -(dsv * wv) * ddv

    _prop_pipeline(y, src_a, dst_a, lap_a, (rows_a, rows_b),
                   (gsem_a, gsem_b), (ssem_a, ssem_b), t_sh, make_lap)

    pltpu.sync_copy(lap_a, lap_out.at[wid])
    plsc.subcore_barrier()
    pltpu.sync_copy(t_sh.at[pl.ds(sid * SLICE, SLICE)],
                    t_out.at[cid, pl.ds(sid * SLICE, SLICE)])


@functools.partial(
    pl.kernel,
    out_type=jax.ShapeDtypeStruct((NC, NPAD, F), jnp.float32),
    mesh=_mesh,
    compiler_params=_params,
    scratch_types=[
        pltpu.VMEM((CPT, CH), jnp.int32),
        pltpu.VMEM((CPT, CH), jnp.int32),
        pltpu.VMEM((CPT, CH), jnp.float32),
        pltpu.VMEM((CH, F), jnp.float32),
        pltpu.VMEM((CH, F), jnp.float32),
        pltpu.SemaphoreType.DMA,
        pltpu.SemaphoreType.DMA,
        pltpu.SemaphoreType.DMA,
        pltpu.SemaphoreType.DMA,
        pltpu.VMEM_SHARED((NPAD, F), jnp.float32),
        pltpu.VMEM_SHARED((N, F), jnp.float32),
    ],
)
def _sc_prop2(ei, lap, y, t_out,
              src_a, dst_a, lap_a, rows_a, rows_b,
              gsem_a, gsem_b, ssem_a, ssem_b, t_sh, y_sh):
    cid = lax.axis_index("c")
    sid = lax.axis_index("s")
    wid = cid * NS + sid

    pltpu.sync_copy(ei.at[0, wid], src_a)
    pltpu.sync_copy(ei.at[1, wid], dst_a)
    pltpu.sync_copy(lap.at[wid], lap_a)

    _zero_acc(rows_a, t_sh, sid)

    @pl.when(sid == 0)
    def _():
        pltpu.sync_copy(y, y_sh)

    plsc.subcore_barrier()

    _prop_pipeline(y_sh, src_a, dst_a, lap_a, (rows_a, rows_b),
                   (gsem_a, gsem_b), (ssem_a, ssem_b), t_sh, lambda c: None)

    plsc.subcore_barrier()
    pltpu.sync_copy(t_sh.at[pl.ds(sid * SLICE, SLICE)],
                    t_out.at[cid, pl.ds(sid * SLICE, SLICE)])


_RB = 2000


def _mm_body(x_ref, w_ref, o_ref):
    o_ref[...] = jnp.dot(x_ref[...], w_ref[...],
                         precision=lax.Precision.HIGHEST,
                         preferred_element_type=jnp.float32)


def _comb_body(b_ref, t_ref, o_ref):
    o_ref[...] = b_ref[...] + 2.0 * (t_ref[0] + t_ref[1])


def _fin_body(a_ref, u_ref, bc_ref, wl_ref, bl_ref, o_ref):
    h = a_ref[...] + u_ref[0] + u_ref[1] + bc_ref[...]
    h = jnp.maximum(h, 0.0)
    o_ref[...] = jnp.dot(h, wl_ref[...],
                         precision=lax.Precision.HIGHEST,
                         preferred_element_type=jnp.float32) + bl_ref[...]


def kernel(x, edge_index, edge_weight, W0, W1, W2, b_cheb, W_lin, b_lin):
    Wc = jnp.concatenate([W0 - W2, W1, W2], axis=1)

    abc = pl.pallas_call(
        _mm_body,
        grid=(N // _RB,),
        in_specs=[pl.BlockSpec((_RB, D), lambda i: (i, 0)),
                  pl.BlockSpec((D, 3 * F), lambda i: (0, 0))],
        out_specs=pl.BlockSpec((_RB, 3 * F), lambda i: (i, 0)),
        out_shape=jax.ShapeDtypeStruct((N, 3 * F), jnp.float32),
    )(x, Wc)
    A = abc[:, :F]
    B = abc[:, F:2 * F]
    C = abc[:, 2 * F:]

    pad = EPAD - E
    ei_p = jnp.pad(edge_index, ((0, 0), (0, pad))).reshape(2, NW, CPT, CH)
    w_p = jnp.pad(edge_weight, ((0, pad),)).reshape(NW, CPT, CH)

    deg_part = _sc_degree(ei_p, w_p)
    t_part, lap = _sc_prop1(ei_p, w_p, deg_part, C)

    M = pl.pallas_call(
        _comb_body,
        grid=(N // _RB,),
        in_specs=[pl.BlockSpec((_RB, F), lambda i: (i, 0)),
                  pl.BlockSpec((NC, _RB, F), lambda i: (0, i, 0))],
        out_specs=pl.BlockSpec((_RB, F), lambda i: (i, 0)),
        out_shape=jax.ShapeDtypeStruct((N, F), jnp.float32),
    )(B, t_part)

    u_part = _sc_prop2(ei_p, lap, M)

    out = pl.pallas_call(
        _fin_body,
        grid=(N // _RB,),
        in_specs=[pl.BlockSpec((_RB, F), lambda i: (i, 0)),
                  pl.BlockSpec((NC, _RB, F), lambda i: (0, i, 0)),
                  pl.BlockSpec((1, F), lambda i: (0, 0)),
                  pl.BlockSpec((F, 1), lambda i: (0, 0)),
                  pl.BlockSpec((1, 1), lambda i: (0, 0))],
        out_specs=pl.BlockSpec((_RB, 1), lambda i: (i, 0)),
        out_shape=jax.ShapeDtypeStruct((N, 1), jnp.float32),
    )(A, u_part, b_cheb.reshape(1, F), W_lin, b_lin.reshape(1, 1))
    return out
